# Initial kernel scaffold; baseline (speedup 1.0000x reference)
import jax
import jax.numpy as jnp
from jax.experimental import pallas as pl


def kernel(query_indices, visual_indices, target_indices, self_attentions, topk_query_indices, global_pattern_per_query, batch_idx, labels):
    raise NotImplementedError("write your pallas kernel here")


# speedup vs baseline: 39.3487x; 39.3487x over previous
import functools

import jax
import jax.numpy as jnp
from jax import lax
from jax.experimental import pallas as pl
from jax.experimental.pallas import tpu as pltpu
from jax.experimental.pallas import tpu_sc as plsc

L, B, H, S, V, QK = 4, 1, 12, 2048, 1024, 8
LH = L * H
EPS = 1e-08
NC, NS, LANES = 2, 16, 16
NW = NC * NS
ROWS = 16


def _gather_body(table, idx_hbm, vis_hbm, out_tgt, out_qsum,
                 idx_v, rows_v, vis_v, tgt_v, acc_v, sem):
    wid = lax.axis_index("s") * NC + lax.axis_index("c")
    pltpu.sync_copy(vis_hbm, vis_v)

    def do_head(m):
        pltpu.sync_copy(idx_hbm.at[m], idx_v)
        pltpu.async_copy(table.at[idx_v], rows_v, sem).wait()

        def chunk(c, qacc):
            off = pl.multiple_of(c * LANES, LANES)
            col = vis_v[pl.ds(off, LANES)]
            acc = qacc
            for j in range(QK):
                row = jnp.full((LANES,), j, jnp.int32)
                acc = acc + plsc.load_gather(rows_v, [row, col])
            trow = jnp.full((LANES,), QK, jnp.int32)
            t = plsc.load_gather(rows_v, [trow, col])
            tgt_v[pl.ds(off, LANES)] = t
            return acc

        qacc = lax.fori_loop(0, V // LANES, chunk,
                             jnp.zeros((LANES,), jnp.float32))
        acc_v[...] = qacc
        pltpu.sync_copy(tgt_v, out_tgt.at[m])
        pltpu.sync_copy(acc_v, out_qsum.at[m])

    do_head(wid)

    @pl.when(wid < LH - NW)
    def _second():
        do_head(wid + NW)


_gather = functools.partial(
    pl.kernel,
    out_type=(jax.ShapeDtypeStruct((LH, V), jnp.float32),
              jax.ShapeDtypeStruct((LH, LANES), jnp.float32)),
    mesh=plsc.VectorSubcoreMesh(core_axis_name="c", subcore_axis_name="s"),
    scratch_types=[
        pltpu.VMEM((ROWS,), jnp.int32),
        pltpu.VMEM((ROWS, S), jnp.float32),
        pltpu.VMEM((V,), jnp.int32),
        pltpu.VMEM((V,), jnp.float32),
        pltpu.VMEM((LANES,), jnp.float32),
        pltpu.SemaphoreType.DMA,
    ],
    compiler_params=pltpu.CompilerParams(needs_layout_passes=False),
)(_gather_body)


def _combine_body(qsum16_ref, tgt_ref, labels_ref, out_m_ref, out_l_ref):
    tgt = tgt_ref[...]
    q = jnp.sum(qsum16_ref[...], axis=1)
    e = jnp.exp(q - jnp.max(q))
    hw = e / jnp.sum(e)
    f = jnp.sum(tgt, axis=1)
    fi = f[:, None]
    fj = f[None, :]
    ii = lax.broadcasted_iota(jnp.int32, (LH, LH), 0)
    jj = lax.broadcasted_iota(jnp.int32, (LH, LH), 1)
    rank = jnp.sum(((fj > fi) | ((fj == fi) & (jj < ii))).astype(jnp.int32),
                   axis=1)
    w = jnp.sum(jnp.where(rank[:, None] == jj, hw[None, :], 0.0), axis=1)
    merged = jnp.sum(w[:, None] * tgt, axis=0)
    merged = merged / (jnp.sum(merged) + EPS)
    lab = labels_ref[...]
    t = lab / (jnp.sum(lab) + EPS)
    pred_log = jnp.log(merged)[None, :]
    safe_t = jnp.where(t > 0, t, 1.0)
    kl = jnp.where(t > 0, t * (jnp.log(safe_t) - pred_log), 0.0)
    out_m_ref[...] = merged[None, :]
    out_l_ref[...] = jnp.sum(kl, keepdims=True)


_combine = pl.pallas_call(
    _combine_body,
    out_shape=(jax.ShapeDtypeStruct((1, V), jnp.float32),
               jax.ShapeDtypeStruct((1, 1), jnp.float32)),
)


def kernel(query_indices, visual_indices, target_indices, self_attentions,
           topk_query_indices, global_pattern_per_query, batch_idx, labels):
    table = self_attentions.reshape(L * H * S, S)
    vis = visual_indices.astype(jnp.int32)
    tq = topk_query_indices.astype(jnp.int32)
    base = jnp.arange(LH, dtype=jnp.int32) * S
    qrows = base[:, None] + tq[None, :]
    trow = base[:, None] + (S - 1)
    pad = jnp.broadcast_to(trow, (LH, ROWS - QK - 1))
    idx = jnp.concatenate([qrows, trow, pad], axis=1)
    tgt, qsum16 = _gather(table, idx, vis)
    merged, loss = _combine(qsum16, tgt, labels.astype(jnp.float32))
    return merged, loss.reshape(())


# speedup vs baseline: 44.2330x; 1.1241x over previous
import functools

import jax
import jax.numpy as jnp
from jax import lax
from jax.experimental import pallas as pl
from jax.experimental.pallas import tpu as pltpu
from jax.experimental.pallas import tpu_sc as plsc

L, B, H, S, V, QK = 4, 1, 12, 2048, 1024, 8
LH = L * H
EPS = 1e-08
NC, NS, LANES = 2, 16, 16
NW = NC * NS
NB = LH - NW
IDXW = 32
CHUNKS = V // LANES


def _gather_body(table, idx_a_hbm, idx_b_hbm, vis_hbm, out_tgt, out_qsum,
                 idx_a_v, idx_b_v, rows_a, rows_b, vis_v, tgt_v, acc_v,
                 sem_a, sem_b):
    wid = lax.axis_index("s") * NC + lax.axis_index("c")
    pltpu.sync_copy(idx_a_hbm.at[wid], idx_a_v)
    pltpu.sync_copy(idx_b_hbm.at[wid], idx_b_v)
    cp_a = pltpu.make_async_copy(table.at[idx_a_v], rows_a, sem_a)
    cp_b = pltpu.make_async_copy(table.at[idx_b_v], rows_b, sem_b)
    cp_a.start()
    cp_b.start()
    pltpu.sync_copy(vis_hbm, vis_v)

    def col_at(c):
        off = pl.multiple_of(c * LANES, LANES)
        return off, vis_v[pl.ds(off, LANES)]

    def gather_row(rows, j, col):
        row = jnp.full((LANES,), j, jnp.int32)
        return plsc.load_gather(rows, [row, col])

    def qsum_out(slot, qacc):
        acc_v[...] = qacc
        pltpu.sync_copy(acc_v, out_qsum.at[slot])

    cp_a.wait()

    def chunk_a(c, qacc):
        _, col = col_at(c)
        acc = qacc
        for j in range(QK):
            acc = acc + gather_row(rows_a, j, col)
        return acc

    qacc = lax.fori_loop(0, CHUNKS, chunk_a, jnp.zeros((LANES,), jnp.float32))
    qsum_out(wid, qacc)

    cp_b.wait()

    def chunk_b(c, qacc):
        _, col = col_at(c)
        acc = qacc
        for j in range(4):
            acc = acc + gather_row(rows_b, j, col)
        return acc

    qacc = lax.fori_loop(0, CHUNKS, chunk_b, jnp.zeros((LANES,), jnp.float32))
    qsum_out(NW + wid, qacc)

    def chunk_t(j, slot):
        def body(c, carry):
            off, col = col_at(c)
            tgt_v[pl.ds(off, LANES)] = gather_row(rows_b, j, col)
            return carry
        lax.fori_loop(0, CHUNKS, body, 0)
        pltpu.sync_copy(tgt_v, out_tgt.at[slot])

    chunk_t(4, wid)

    @pl.when(wid >= NS)
    def _tgt_b():
        chunk_t(5, NW + wid - NS)


_gather = functools.partial(
    pl.kernel,
    out_type=(jax.ShapeDtypeStruct((LH, V), jnp.float32),
              jax.ShapeDtypeStruct((2 * NW, LANES), jnp.float32)),
    mesh=plsc.VectorSubcoreMesh(core_axis_name="c", subcore_axis_name="s"),
    scratch_types=[
        pltpu.VMEM((QK,), jnp.int32),
        pltpu.VMEM((QK,), jnp.int32),
        pltpu.VMEM((QK, S), jnp.float32),
        pltpu.VMEM((QK, S), jnp.float32),
        pltpu.VMEM((V,), jnp.int32),
        pltpu.VMEM((V,), jnp.float32),
        pltpu.VMEM((LANES,), jnp.float32),
        pltpu.SemaphoreType.DMA,
        pltpu.SemaphoreType.DMA,
    ],
    compiler_params=pltpu.CompilerParams(needs_layout_passes=False),
)(_gather_body)


def _combine_body(qsum16_ref, tgt_ref, labels_ref, out_m_ref, out_l_ref):
    tgt = tgt_ref[...]
    qraw = jnp.sum(qsum16_ref[...], axis=1)
    q = jnp.concatenate([qraw[:NW], qraw[NW:NW + NB] + qraw[NW + NB:]])
    e = jnp.exp(q - jnp.max(q))
    hw = e / jnp.sum(e)
    f = jnp.sum(tgt, axis=1)
    fi = f[:, None]
    fj = f[None, :]
    ii = lax.broadcasted_iota(jnp.int32, (LH, LH), 0)
    jj = lax.broadcasted_iota(jnp.int32, (LH, LH), 1)
    rank = jnp.sum(((fj > fi) | ((fj == fi) & (jj < ii))).astype(jnp.int32),
                   axis=1)
    w = jnp.sum(jnp.where(rank[:, None] == jj, hw[None, :], 0.0), axis=1)
    merged = jnp.sum(w[:, None] * tgt, axis=0)
    merged = merged / (jnp.sum(merged) + EPS)
    lab = labels_ref[...]
    t = lab / (jnp.sum(lab) + EPS)
    pred_log = jnp.log(merged)[None, :]
    safe_t = jnp.where(t > 0, t, 1.0)
    kl = jnp.where(t > 0, t * (jnp.log(safe_t) - pred_log), 0.0)
    out_m_ref[...] = merged[None, :]
    out_l_ref[...] = jnp.sum(kl, keepdims=True)


_combine = pl.pallas_call(
    _combine_body,
    out_shape=(jax.ShapeDtypeStruct((1, V), jnp.float32),
               jax.ShapeDtypeStruct((1, 1), jnp.float32)),
)


def kernel(query_indices, visual_indices, target_indices, self_attentions,
           topk_query_indices, global_pattern_per_query, batch_idx, labels):
    table = self_attentions.reshape(L * H * S, S)
    vis = visual_indices.astype(jnp.int32)
    tq = topk_query_indices.astype(jnp.int32)
    w = jnp.arange(NW, dtype=jnp.int32)
    base_a = (w * S)[:, None]
    base_b = ((NW + w % NS) * S)[:, None]
    idx_a = base_a + tq[None, :]
    half = jnp.where(w >= NS, 4, 0)[:, None]
    tgt_a = base_a + (S - 1)
    tgt_b = base_b + (S - 1)
    idx_b = jnp.concatenate(
        [base_b + jnp.take(tq, jnp.arange(4)[None, :] + half, axis=0),
         tgt_a, tgt_b, tgt_a, tgt_a], axis=1)
    tgt, qsum16 = _gather(table, idx_a, idx_b, vis)
    merged, loss = _combine(qsum16, tgt, labels.astype(jnp.float32))
    return merged, loss.reshape(())


# speedup vs baseline: 46.8044x; 1.0581x over previous
import functools

import jax
import jax.numpy as jnp
from jax import lax
from jax.experimental import pallas as pl
from jax.experimental.pallas import tpu as pltpu
from jax.experimental.pallas import tpu_sc as plsc

L, B, H, S, V, QK = 4, 1, 12, 2048, 1024, 8
LH = L * H
EPS = 1e-08
NC, NS, LANES = 2, 16, 16
NW = NC * NS
NB = LH - NW
IDXW = 32
CHUNKS = V // LANES


def _gather_body(table, idx_a_hbm, idx_b_hbm, vis_hbm, out_tgt, out_qsum,
                 idx_a_v, idx_b_v, rows_a, rows_b, vis_v, tgt_v, tgt_v2,
                 acc_v, sem_a, sem_b):
    wid = lax.axis_index("s") * NC + lax.axis_index("c")
    pltpu.sync_copy(idx_a_hbm.at[wid], idx_a_v)
    pltpu.sync_copy(idx_b_hbm.at[wid], idx_b_v)
    cp_a = pltpu.make_async_copy(table.at[idx_a_v], rows_a, sem_a)
    cp_b = pltpu.make_async_copy(table.at[idx_b_v], rows_b, sem_b)
    cp_a.start()
    cp_b.start()
    pltpu.sync_copy(vis_hbm, vis_v)

    def col_at(c):
        off = pl.multiple_of(c * LANES, LANES)
        return off, vis_v[pl.ds(off, LANES)]

    def gather_row(rows, j, col):
        row = jnp.full((LANES,), j, jnp.int32)
        return plsc.load_gather(rows, [row, col])

    def qsum_out(slot, qacc):
        acc_v[...] = qacc
        pltpu.sync_copy(acc_v, out_qsum.at[slot])

    zero = jnp.zeros((LANES,), jnp.float32)

    cp_a.wait()

    @plsc.parallel_loop(0, CHUNKS, unroll=2, carry=zero)
    def qacc_a(c, acc):
        _, col = col_at(c)
        g = [gather_row(rows_a, j, col) for j in range(QK)]
        return acc + (((g[0] + g[1]) + (g[2] + g[3]))
                      + ((g[4] + g[5]) + (g[6] + g[7])))

    qsum_out(wid, qacc_a)

    cp_b.wait()

    @plsc.parallel_loop(0, CHUNKS, unroll=2, carry=zero)
    def qacc_b(c, acc):
        off, col = col_at(c)
        g = [gather_row(rows_b, j, col) for j in range(4)]
        tgt_v[pl.ds(off, LANES)] = gather_row(rows_b, 4, col)
        tgt_v2[pl.ds(off, LANES)] = gather_row(rows_b, 5, col)
        return acc + ((g[0] + g[1]) + (g[2] + g[3]))

    qsum_out(NW + wid, qacc_b)
    pltpu.sync_copy(tgt_v, out_tgt.at[wid])

    @pl.when(wid >= NS)
    def _tgt_b():
        pltpu.sync_copy(tgt_v2, out_tgt.at[NW + wid - NS])


_gather = functools.partial(
    pl.kernel,
    out_type=(jax.ShapeDtypeStruct((LH, V), jnp.float32),
              jax.ShapeDtypeStruct((2 * NW, LANES), jnp.float32)),
    mesh=plsc.VectorSubcoreMesh(core_axis_name="c", subcore_axis_name="s"),
    scratch_types=[
        pltpu.VMEM((QK,), jnp.int32),
        pltpu.VMEM((QK,), jnp.int32),
        pltpu.VMEM((QK, S), jnp.float32),
        pltpu.VMEM((QK, S), jnp.float32),
        pltpu.VMEM((V,), jnp.int32),
        pltpu.VMEM((V,), jnp.float32),
        pltpu.VMEM((V,), jnp.float32),
        pltpu.VMEM((LANES,), jnp.float32),
        pltpu.SemaphoreType.DMA,
        pltpu.SemaphoreType.DMA,
    ],
    compiler_params=pltpu.CompilerParams(needs_layout_passes=False),
)(_gather_body)


def _combine_body(qsum16_ref, tgt_ref, labels_ref, out_m_ref, out_l_ref):
    tgt = tgt_ref[...]
    qraw = jnp.sum(qsum16_ref[...], axis=1)
    q = jnp.concatenate([qraw[:NW], qraw[NW:NW + NB] + qraw[NW + NB:]])
    e = jnp.exp(q - jnp.max(q))
    hw = e / jnp.sum(e)
    f = jnp.sum(tgt, axis=1)
    fi = f[:, None]
    fj = f[None, :]
    ii = lax.broadcasted_iota(jnp.int32, (LH, LH), 0)
    jj = lax.broadcasted_iota(jnp.int32, (LH, LH), 1)
    rank = jnp.sum(((fj > fi) | ((fj == fi) & (jj < ii))).astype(jnp.int32),
                   axis=1)
    w = jnp.sum(jnp.where(rank[:, None] == jj, hw[None, :], 0.0), axis=1)
    merged = jnp.sum(w[:, None] * tgt, axis=0)
    merged = merged / (jnp.sum(merged) + EPS)
    lab = labels_ref[...]
    t = lab / (jnp.sum(lab) + EPS)
    pred_log = jnp.log(merged)[None, :]
    safe_t = jnp.where(t > 0, t, 1.0)
    kl = jnp.where(t > 0, t * (jnp.log(safe_t) - pred_log), 0.0)
    out_m_ref[...] = merged[None, :]
    out_l_ref[...] = jnp.sum(kl, keepdims=True)


_combine = pl.pallas_call(
    _combine_body,
    out_shape=(jax.ShapeDtypeStruct((1, V), jnp.float32),
               jax.ShapeDtypeStruct((1, 1), jnp.float32)),
)


def kernel(query_indices, visual_indices, target_indices, self_attentions,
           topk_query_indices, global_pattern_per_query, batch_idx, labels):
    table = self_attentions.reshape(L * H * S, S)
    vis = visual_indices.astype(jnp.int32)
    tq = topk_query_indices.astype(jnp.int32)
    w = jnp.arange(NW, dtype=jnp.int32)
    base_a = (w * S)[:, None]
    base_b = ((NW + w % NS) * S)[:, None]
    idx_a = base_a + tq[None, :]
    hi = (w >= NS)[:, None]
    bq = jnp.where(hi, tq[None, 4:], tq[None, :4])
    tgt_a = base_a + (S - 1)
    tgt_b = base_b + (S - 1)
    idx_b = jnp.concatenate(
        [base_b + bq, tgt_a, tgt_b, tgt_a, tgt_a], axis=1)
    tgt, qsum16 = _gather(table, idx_a, idx_b, vis)
    merged, loss = _combine(qsum16, tgt, labels.astype(jnp.float32))
    return merged, loss.reshape(())


# speedup vs baseline: 47.8640x; 1.0226x over previous
import functools

import jax
import jax.numpy as jnp
from jax import lax
from jax.experimental import pallas as pl
from jax.experimental.pallas import tpu as pltpu
from jax.experimental.pallas import tpu_sc as plsc

L, B, H, S, V, QK = 4, 1, 12, 2048, 1024, 8
LH = L * H
EPS = 1e-08
NC, NS, LANES = 2, 16, 16
NW = NC * NS
NB = LH - NW
IDXW = 32
CHUNKS = V // LANES


def _gather_body(table, tq_hbm, vis_hbm, out_tgt, out_qsum,
                 tq16_v, idx_a_v, idx_b_v, rows_a, rows_b, vis_v, tgt_v,
                 tgt_v2, acc_v, sem_a, sem_b):
    wid = lax.axis_index("s") * NC + lax.axis_index("c")
    wid_s = wid * S
    b_s = (NW + lax.rem(wid, NS)) * S
    half = jnp.where(wid >= NS, 4, 0)
    pltpu.sync_copy(tq_hbm, tq16_v.at[pl.ds(0, QK)])
    tql = tq16_v[...]
    lane = lax.iota(jnp.int32, 16)
    pos = jnp.where(lane < QK, lane,
                    jnp.where(lane < 12, half + lane - QK, 0))
    vals = tql.at[pos].get(mode="promise_in_bounds")
    base = jnp.where((lane < QK) | (lane == 12), wid_s, b_s)
    idxv = base + jnp.where(lane < 12, vals, S - 1)
    store_scatter = plsc.store_scatter
    store_scatter(idx_a_v, [jnp.where(lane < QK, lane, 0)], idxv,
                  mask=lane < QK)
    store_scatter(idx_b_v, [jnp.clip(lane - QK, 0, 5)], idxv,
                  mask=(lane >= QK) & (lane < 14))
    cp_a = pltpu.make_async_copy(table.at[idx_a_v], rows_a, sem_a)
    cp_b = pltpu.make_async_copy(table.at[idx_b_v], rows_b, sem_b)
    cp_a.start()
    cp_b.start()
    pltpu.sync_copy(vis_hbm, vis_v)

    def col_at(c):
        off = pl.multiple_of(c * LANES, LANES)
        return off, vis_v[pl.ds(off, LANES)]

    def gather_row(rows, j, col):
        row = jnp.full((LANES,), j, jnp.int32)
        return plsc.load_gather(rows, [row, col])

    def qsum_out(slot, qacc):
        acc_v[...] = qacc
        pltpu.sync_copy(acc_v, out_qsum.at[slot])

    zero = jnp.zeros((LANES,), jnp.float32)

    cp_a.wait()

    @plsc.parallel_loop(0, CHUNKS, unroll=2, carry=zero)
    def qacc_a(c, acc):
        _, col = col_at(c)
        g = [gather_row(rows_a, j, col) for j in range(QK)]
        return acc + (((g[0] + g[1]) + (g[2] + g[3]))
                      + ((g[4] + g[5]) + (g[6] + g[7])))

    qsum_out(wid, qacc_a)

    cp_b.wait()

    @plsc.parallel_loop(0, CHUNKS, unroll=2, carry=zero)
    def qacc_b(c, acc):
        off, col = col_at(c)
        g = [gather_row(rows_b, j, col) for j in range(4)]
        tgt_v[pl.ds(off, LANES)] = gather_row(rows_b, 4, col)
        tgt_v2[pl.ds(off, LANES)] = gather_row(rows_b, 5, col)
        return acc + ((g[0] + g[1]) + (g[2] + g[3]))

    qsum_out(NW + wid, qacc_b)
    pltpu.sync_copy(tgt_v, out_tgt.at[wid])

    @pl.when(wid >= NS)
    def _tgt_b():
        pltpu.sync_copy(tgt_v2, out_tgt.at[NW + wid - NS])


_gather = functools.partial(
    pl.kernel,
    out_type=(jax.ShapeDtypeStruct((LH, V), jnp.float32),
              jax.ShapeDtypeStruct((2 * NW, LANES), jnp.float32)),
    mesh=plsc.VectorSubcoreMesh(core_axis_name="c", subcore_axis_name="s"),
    scratch_types=[
        pltpu.VMEM((LANES,), jnp.int32),
        pltpu.VMEM((QK,), jnp.int32),
        pltpu.VMEM((6,), jnp.int32),
        pltpu.VMEM((QK, S), jnp.float32),
        pltpu.VMEM((6, S), jnp.float32),
        pltpu.VMEM((V,), jnp.int32),
        pltpu.VMEM((V,), jnp.float32),
        pltpu.VMEM((V,), jnp.float32),
        pltpu.VMEM((LANES,), jnp.float32),
        pltpu.SemaphoreType.DMA,
        pltpu.SemaphoreType.DMA,
    ],
    compiler_params=pltpu.CompilerParams(needs_layout_passes=False),
)(_gather_body)


def _combine_body(qsum16_ref, tgt_ref, labels_ref, out_m_ref, out_l_ref):
    tgt = tgt_ref[...]
    qraw = jnp.sum(qsum16_ref[...], axis=1)
    q = jnp.concatenate([qraw[:NW], qraw[NW:NW + NB] + qraw[NW + NB:]])
    e = jnp.exp(q - jnp.max(q))
    hw = e / jnp.sum(e)
    f = jnp.sum(tgt, axis=1)
    fi = f[:, None]
    fj = f[None, :]
    ii = lax.broadcasted_iota(jnp.int32, (LH, LH), 0)
    jj = lax.broadcasted_iota(jnp.int32, (LH, LH), 1)
    rank = jnp.sum(((fj > fi) | ((fj == fi) & (jj < ii))).astype(jnp.int32),
                   axis=1)
    w = jnp.sum(jnp.where(rank[:, None] == jj, hw[None, :], 0.0), axis=1)
    merged = jnp.sum(w[:, None] * tgt, axis=0)
    merged = merged / (jnp.sum(merged) + EPS)
    lab = labels_ref[...]
    t = lab / (jnp.sum(lab) + EPS)
    pred_log = jnp.log(merged)[None, :]
    safe_t = jnp.where(t > 0, t, 1.0)
    kl = jnp.where(t > 0, t * (jnp.log(safe_t) - pred_log), 0.0)
    out_m_ref[...] = merged[None, :]
    out_l_ref[...] = jnp.sum(kl, keepdims=True)


_combine = pl.pallas_call(
    _combine_body,
    out_shape=(jax.ShapeDtypeStruct((1, V), jnp.float32),
               jax.ShapeDtypeStruct((1, 1), jnp.float32)),
)


def kernel(query_indices, visual_indices, target_indices, self_attentions,
           topk_query_indices, global_pattern_per_query, batch_idx, labels):
    table = self_attentions.reshape(L * H * S, S)
    vis = visual_indices.astype(jnp.int32)
    tq = topk_query_indices.astype(jnp.int32)
    tgt, qsum16 = _gather(table, tq, vis)
    merged, loss = _combine(qsum16, tgt, labels.astype(jnp.float32))
    return merged, loss.reshape(())


# speedup vs baseline: 48.1474x; 1.0059x over previous
import functools

import jax
import jax.numpy as jnp
from jax import lax
from jax.experimental import pallas as pl
from jax.experimental.pallas import tpu as pltpu
from jax.experimental.pallas import tpu_sc as plsc

L, B, H, S, V, QK = 4, 1, 12, 2048, 1024, 8
LH = L * H
EPS = 1e-08
NC, NS, LANES = 2, 16, 16
NW = NC * NS
NB = LH - NW
IDXW = 32
CHUNKS = V // LANES


def _gather_body(table, tq_hbm, vis_hbm, out_tgt, out_qsum,
                 tq16_v, idx_a_v, idx_b_v, rows_a, rows_b, vis_v, tgt_v,
                 tgt_v2, acc_v, sem_a, sem_b):
    wid = lax.axis_index("s") * NC + lax.axis_index("c")
    wid_s = wid * S
    b_s = (NW + lax.rem(wid, NS)) * S
    half = jnp.where(wid >= NS, 4, 0)
    pltpu.sync_copy(tq_hbm, tq16_v.at[pl.ds(0, QK)])
    tql = tq16_v[...]
    lane = lax.iota(jnp.int32, 16)
    pos = jnp.where(lane < QK, lane,
                    jnp.where(lane < 12, half + lane - QK, 0))
    vals = tql.at[pos].get(mode="promise_in_bounds")
    base = jnp.where((lane < QK) | (lane == 12), wid_s, b_s)
    idxv = base + jnp.where(lane < 12, vals, S - 1)
    store_scatter = plsc.store_scatter
    store_scatter(idx_a_v, [jnp.where(lane < QK, lane, 0)], idxv,
                  mask=lane < QK)
    store_scatter(idx_b_v, [jnp.clip(lane - QK, 0, QK - 1)], idxv,
                  mask=lane >= QK)
    cp_a = pltpu.make_async_copy(table.at[idx_a_v], rows_a, sem_a)
    cp_b = pltpu.make_async_copy(table.at[idx_b_v], rows_b, sem_b)
    cp_a.start()
    cp_b.start()
    pltpu.sync_copy(vis_hbm, vis_v)

    def col_at(c):
        off = pl.multiple_of(c * LANES, LANES)
        return off, vis_v[pl.ds(off, LANES)]

    def gather_row(rows, j, col):
        row = jnp.full((LANES,), j, jnp.int32)
        return plsc.load_gather(rows, [row, col])

    def qsum_out(slot, qacc):
        acc_v[...] = qacc
        pltpu.sync_copy(acc_v, out_qsum.at[slot])

    zero = jnp.zeros((LANES,), jnp.float32)

    cp_a.wait()

    @plsc.parallel_loop(0, CHUNKS, unroll=2, carry=zero)
    def qacc_a(c, acc):
        _, col = col_at(c)
        g = [gather_row(rows_a, j, col) for j in range(QK)]
        return acc + (((g[0] + g[1]) + (g[2] + g[3]))
                      + ((g[4] + g[5]) + (g[6] + g[7])))

    qsum_out(wid, qacc_a)

    cp_b.wait()

    @plsc.parallel_loop(0, CHUNKS, unroll=2, carry=zero)
    def qacc_b(c, acc):
        off, col = col_at(c)
        g = [gather_row(rows_b, j, col) for j in range(4)]
        tgt_v[pl.ds(off, LANES)] = gather_row(rows_b, 4, col)
        tgt_v2[pl.ds(off, LANES)] = gather_row(rows_b, 5, col)
        return acc + ((g[0] + g[1]) + (g[2] + g[3]))

    qsum_out(NW + wid, qacc_b)
    pltpu.sync_copy(tgt_v, out_tgt.at[wid])

    @pl.when(wid >= NS)
    def _tgt_b():
        pltpu.sync_copy(tgt_v2, out_tgt.at[NW + wid - NS])


_gather = functools.partial(
    pl.kernel,
    out_type=(jax.ShapeDtypeStruct((LH, V), jnp.float32),
              jax.ShapeDtypeStruct((2 * NW, LANES), jnp.float32)),
    mesh=plsc.VectorSubcoreMesh(core_axis_name="c", subcore_axis_name="s"),
    scratch_types=[
        pltpu.VMEM((LANES,), jnp.int32),
        pltpu.VMEM((QK,), jnp.int32),
        pltpu.VMEM((QK,), jnp.int32),
        pltpu.VMEM((QK, S), jnp.float32),
        pltpu.VMEM((QK, S), jnp.float32),
        pltpu.VMEM((V,), jnp.int32),
        pltpu.VMEM((V,), jnp.float32),
        pltpu.VMEM((V,), jnp.float32),
        pltpu.VMEM((LANES,), jnp.float32),
        pltpu.SemaphoreType.DMA,
        pltpu.SemaphoreType.DMA,
    ],
    compiler_params=pltpu.CompilerParams(needs_layout_passes=False),
)(_gather_body)


def _combine_body(qsum16_ref, tgt_ref, labels_ref, out_m_ref, out_l_ref):
    tgt = tgt_ref[...]
    qraw = jnp.sum(qsum16_ref[...], axis=1)
    q = jnp.concatenate([qraw[:NW], qraw[NW:NW + NB] + qraw[NW + NB:]])
    e = jnp.exp(q - jnp.max(q))
    hw = e / jnp.sum(e)
    f = jnp.sum(tgt, axis=1)
    fi = f[:, None]
    fj = f[None, :]
    ii = lax.broadcasted_iota(jnp.int32, (LH, LH), 0)
    jj = lax.broadcasted_iota(jnp.int32, (LH, LH), 1)
    rank = jnp.sum(((fj > fi) | ((fj == fi) & (jj < ii))).astype(jnp.int32),
                   axis=1)
    w = jnp.sum(jnp.where(rank[:, None] == jj, hw[None, :], 0.0), axis=1)
    merged = jnp.sum(w[:, None] * tgt, axis=0)
    merged = merged / (jnp.sum(merged) + EPS)
    lab = labels_ref[...]
    t = lab / (jnp.sum(lab) + EPS)
    pred_log = jnp.log(merged)[None, :]
    safe_t = jnp.where(t > 0, t, 1.0)
    kl = jnp.where(t > 0, t * (jnp.log(safe_t) - pred_log), 0.0)
    out_m_ref[...] = merged[None, :]
    out_l_ref[...] = jnp.sum(kl, keepdims=True)


_combine = pl.pallas_call(
    _combine_body,
    out_shape=(jax.ShapeDtypeStruct((1, V), jnp.float32),
               jax.ShapeDtypeStruct((1, 1), jnp.float32)),
)


def kernel(query_indices, visual_indices, target_indices, self_attentions,
           topk_query_indices, global_pattern_per_query, batch_idx, labels):
    table = self_attentions.reshape(L * H * S, S)
    vis = visual_indices.astype(jnp.int32)
    tq = topk_query_indices.astype(jnp.int32)
    tgt, qsum16 = _gather(table, tq, vis)
    merged, loss = _combine(qsum16, tgt, labels.astype(jnp.float32))
    return merged, loss.reshape(())


# speedup vs baseline: 48.6471x; 1.0104x over previous
import functools

import jax
import jax.numpy as jnp
from jax import lax
from jax.experimental import pallas as pl
from jax.experimental.pallas import tpu as pltpu
from jax.experimental.pallas import tpu_sc as plsc

L, B, H, S, V, QK = 4, 1, 12, 2048, 1024, 8
LH = L * H
EPS = 1e-08
NC, NS, LANES = 2, 16, 16
NW = NC * NS
NB = LH - NW
IDXW = 32
CHUNKS = V // LANES


def _gather_body(table, tq_hbm, vis_hbm, out_tgt, out_qsum,
                 tq16_v, idx_a_v, idx_b_v, rows_a, rows_b, vis_v, tgt_v,
                 tgt_v2, acc_v, sem_a, sem_b):
    wid = lax.axis_index("s") * NC + lax.axis_index("c")
    wid_s = wid * S
    b_s = (NW + lax.rem(wid, NS)) * S
    half = jnp.where(wid >= NS, 4, 0)
    pltpu.sync_copy(tq_hbm, tq16_v.at[pl.ds(0, QK)])
    tql = tq16_v[...]
    lane = lax.iota(jnp.int32, 16)
    pos = jnp.where(lane < QK, lane,
                    jnp.where(lane < 12, half + lane - QK, 0))
    vals = tql.at[pos].get(mode="promise_in_bounds")
    base = jnp.where((lane < QK) | (lane == 12), wid_s, b_s)
    idxv = base + jnp.where(lane < 12, vals, S - 1)
    store_scatter = plsc.store_scatter
    store_scatter(idx_a_v, [jnp.where(lane < QK, lane, 0)], idxv,
                  mask=lane < QK)
    store_scatter(idx_b_v, [jnp.clip(lane - QK, 0, QK - 1)], idxv,
                  mask=lane >= QK)
    cp_a = pltpu.make_async_copy(table.at[idx_a_v], rows_a, sem_a)
    cp_b = pltpu.make_async_copy(table.at[idx_b_v], rows_b, sem_b)
    cp_a.start()
    cp_b.start()
    pltpu.sync_copy(vis_hbm, vis_v)

    def col_at(c):
        off = pl.multiple_of(c * LANES, LANES)
        return off, vis_v[pl.ds(off, LANES)]

    def gather_row(rows, j, col):
        row = jnp.full((LANES,), j, jnp.int32)
        return plsc.load_gather(rows, [row, col])

    def qsum_out(slot, qacc):
        acc_v[...] = qacc
        pltpu.sync_copy(acc_v, out_qsum.at[slot])

    zero = jnp.zeros((LANES,), jnp.float32)

    cp_a.wait()

    @plsc.parallel_loop(0, CHUNKS, unroll=2, carry=zero)
    def qacc_a(c, acc):
        _, col = col_at(c)
        g = [gather_row(rows_a, j, col) for j in range(QK)]
        return acc + (((g[0] + g[1]) + (g[2] + g[3]))
                      + ((g[4] + g[5]) + (g[6] + g[7])))

    qsum_out(wid, qacc_a)

    cp_b.wait()

    @plsc.parallel_loop(0, CHUNKS, unroll=2, carry=zero)
    def qacc_b(c, acc):
        off, col = col_at(c)
        g = [gather_row(rows_b, j, col) for j in range(4)]
        tgt_v[pl.ds(off, LANES)] = gather_row(rows_b, 4, col)
        tgt_v2[pl.ds(off, LANES)] = gather_row(rows_b, 5, col)
        return acc + ((g[0] + g[1]) + (g[2] + g[3]))

    qsum_out(NW + wid, qacc_b)
    pltpu.sync_copy(tgt_v, out_tgt.at[wid])

    @pl.when(wid >= NS)
    def _tgt_b():
        pltpu.sync_copy(tgt_v2, out_tgt.at[NW + wid - NS])


_gather = functools.partial(
    pl.kernel,
    out_type=(jax.ShapeDtypeStruct((LH, V), jnp.float32),
              jax.ShapeDtypeStruct((2 * NW, LANES), jnp.float32)),
    mesh=plsc.VectorSubcoreMesh(core_axis_name="c", subcore_axis_name="s"),
    scratch_types=[
        pltpu.VMEM((LANES,), jnp.int32),
        pltpu.VMEM((QK,), jnp.int32),
        pltpu.VMEM((QK,), jnp.int32),
        pltpu.VMEM((QK, S), jnp.float32),
        pltpu.VMEM((QK, S), jnp.float32),
        pltpu.VMEM((V,), jnp.int32),
        pltpu.VMEM((V,), jnp.float32),
        pltpu.VMEM((V,), jnp.float32),
        pltpu.VMEM((LANES,), jnp.float32),
        pltpu.SemaphoreType.DMA,
        pltpu.SemaphoreType.DMA,
    ],
    compiler_params=pltpu.CompilerParams(needs_layout_passes=False),
)(_gather_body)


def _combine_body(qsum16_ref, tgt_ref, labels_ref, out_m_ref, out_l_ref):
    tgt = tgt_ref[...]
    q64 = jnp.sum(qsum16_ref[...], axis=1, keepdims=True)
    q = jnp.concatenate([q64[:NW], q64[NW:NW + NB] + q64[NW + NB:]], axis=0)
    e = jnp.exp(q - jnp.max(q))
    hw = e / jnp.sum(e)
    ii = lax.broadcasted_iota(jnp.int32, (LH, LH), 0)
    jj = lax.broadcasted_iota(jnp.int32, (LH, LH), 1)
    eye = (ii == jj).astype(jnp.float32)
    ones_row = jnp.ones((1, LH), jnp.float32)
    f_col = jnp.sum(tgt, axis=1, keepdims=True)
    f_row = jnp.dot(ones_row, f_col * eye,
                    precision=lax.Precision.HIGHEST)
    cmp = (f_row > f_col) | ((f_row == f_col) & (jj < ii))
    rank = jnp.sum(cmp.astype(jnp.float32), axis=1, keepdims=True)
    hw_row = jnp.dot(ones_row, hw * eye,
                     precision=lax.Precision.HIGHEST)
    w = jnp.sum(jnp.where(rank == jj.astype(jnp.float32), hw_row, 0.0),
                axis=1, keepdims=True)
    merged = jnp.sum(w * tgt, axis=0, keepdims=True)
    merged = merged / (jnp.sum(merged) + EPS)
    lab = labels_ref[...]
    t = lab / (jnp.sum(lab) + EPS)
    pred_log = jnp.log(merged)
    safe_t = jnp.where(t > 0, t, 1.0)
    kl = jnp.where(t > 0, t * (jnp.log(safe_t) - pred_log), 0.0)
    out_m_ref[...] = merged
    out_l_ref[...] = jnp.sum(kl, keepdims=True)


_combine = pl.pallas_call(
    _combine_body,
    out_shape=(jax.ShapeDtypeStruct((1, V), jnp.float32),
               jax.ShapeDtypeStruct((1, 1), jnp.float32)),
)


def kernel(query_indices, visual_indices, target_indices, self_attentions,
           topk_query_indices, global_pattern_per_query, batch_idx, labels):
    table = self_attentions.reshape(L * H * S, S)
    vis = visual_indices.astype(jnp.int32)
    tq = topk_query_indices.astype(jnp.int32)
    tgt, qsum16 = _gather(table, tq, vis)
    merged, loss = _combine(qsum16, tgt, labels.astype(jnp.float32))
    return merged, loss.reshape(())
